# baseline (speedup 1.0000x reference)
import jax
import jax.numpy as jnp
from jax import lax
from jax.experimental import pallas as pl
from jax.experimental.pallas import tpu as pltpu
import jax.experimental.pallas.tpu_sc as plsc

Q = 2048
K = 2048
H = 16
NSHIFT = 8
TWC = 257
TW = TWC * 16
UU = 4
GROUPS = Q // NSHIFT

_THRESH = (12, 16, 23, 32, 46, 64, 91)


def _sc_body(bias_hbm, tbl_hbm, bias_v, tbl_v, sem):
    c = lax.axis_index("c")
    s = lax.axis_index("s")
    h = s

    pltpu.sync_copy(bias_hbm, bias_v)

    lane = lax.iota(jnp.int32, 16)
    hvec = jnp.full((16,), h, jnp.int32)

    for uu in range(UU):
        ushift = 7 - uu - (Q - 1)

        def build(t, carry, _ushift=ushift, _uu=uu):
            d = t * 16 + (_ushift - 4 * c) + lane
            pos = d > 0
            a = jnp.abs(d)
            large = jnp.full((16,), 8, jnp.int32)
            for idx, thr in enumerate(_THRESH):
                large = jnp.where(a >= thr, 9 + idx, large)
            sub = jnp.where(a < 8, a, large)
            bkt = jnp.where(pos, sub + 16, sub)
            vals = plsc.load_gather(bias_v, [bkt, hvec])
            tbl_v[pl.ds((_uu * TWC + t) * 16, 16)] = vals
            return carry

        lax.fori_loop(0, TWC, build, 0)

    wid = h * 2 + c
    pltpu.async_copy(
        tbl_v, tbl_hbm.at[pl.ds(wid * (UU * TW), UU * TW)], sem
    ).wait()


def _tc_body(tbl_ref, out_ref):
    q = pl.program_id(1)
    base = (Q - NSHIFT) - NSHIFT * q
    base_al = (base // 128) * 128
    r = base - base_al
    win = tbl_ref[0, :, pl.ds(base_al, K + 128)]
    rolled = pltpu.roll(win, ((K + 128) - r) % (K + 128), 1)
    out_ref[0, 0, :, :] = rolled[:, :K]


@jax.jit
def _bias_grid(relative_attention_bias):
    mesh = plsc.VectorSubcoreMesh(
        core_axis_name="c", subcore_axis_name="s", num_cores=2, num_subcores=16
    )
    tbl_flat = pl.kernel(
        _sc_body,
        out_type=jax.ShapeDtypeStruct((H * NSHIFT * TW,), jnp.float32),
        mesh=mesh,
        scratch_types=[
            pltpu.VMEM((32, H), jnp.float32),
            pltpu.VMEM((UU * TW,), jnp.float32),
            pltpu.SemaphoreType.DMA,
        ],
        compiler_params=pltpu.CompilerParams(needs_layout_passes=False),
    )(relative_attention_bias)

    tbl3 = tbl_flat.reshape(H, NSHIFT, TW)

    return pl.pallas_call(
        _tc_body,
        grid=(H, GROUPS),
        in_specs=[
            pl.BlockSpec((1, NSHIFT, TW), lambda hh, qq: (hh, 0, 0)),
        ],
        out_specs=pl.BlockSpec(
            (1, 1, NSHIFT, K), lambda hh, qq: (0, hh, qq, 0)
        ),
        out_shape=jax.ShapeDtypeStruct((1, H, Q, K), jnp.float32),
    )(tbl3)


def kernel(encoder_hidden, decoder_hidden, relative_attention_bias):
    return _bias_grid(relative_attention_bias)


# speedup vs baseline: 1.0477x; 1.0477x over previous
import jax
import jax.numpy as jnp
from jax import lax
from jax.experimental import pallas as pl
from jax.experimental.pallas import tpu as pltpu
import jax.experimental.pallas.tpu_sc as plsc

Q = 2048
K = 2048
H = 16
NSHIFT = 8
NPHASE = 16
TWC = 256
TW = TWC * 16
TWP = 3968
UU = 4
GROUPS = Q // NSHIFT

_THRESH = (12, 16, 23, 32, 46, 64, 91)


def _sc_body(bias_hbm, var_hbm, bias_v, tbl_v, sem):
    c = lax.axis_index("c")
    s = lax.axis_index("s")
    h = s

    pltpu.sync_copy(bias_hbm, bias_v)

    lane = lax.iota(jnp.int32, 16)
    hvec = jnp.full((16,), h, jnp.int32)

    for uu in range(UU):
        ushift = 7 - uu - (Q - 1)

        def build(t, carry, _ushift=ushift, _uu=uu):
            d = t * 16 + (_ushift - 4 * c) + lane
            pos = d > 0
            a = jnp.abs(d)
            large = jnp.full((16,), 8, jnp.int32)
            for idx, thr in enumerate(_THRESH):
                large = jnp.where(a >= thr, 9 + idx, large)
            sub = jnp.where(a < 8, a, large)
            bkt = jnp.where(pos, sub + 16, sub)
            vals = plsc.load_gather(bias_v, [bkt, hvec])
            tbl_v[pl.ds((_uu * TWC + t) * 16, 16)] = vals
            return carry

        lax.fori_loop(0, TWC, build, 0)

    for uu in range(UU):
        for m in range(NPHASE):
            src = tbl_v.at[pl.ds(uu * TW + (120 - 8 * m), TWP)]
            dst_off = ((h * NPHASE + m) * NSHIFT + 4 * c + uu) * TWP
            pltpu.async_copy(src, var_hbm.at[pl.ds(dst_off, TWP)], sem)
    for _ in range(UU * NPHASE):
        pltpu.make_async_copy(
            var_hbm.at[pl.ds(0, TWP)], tbl_v.at[pl.ds(0, TWP)], sem
        ).wait()


def _tc_body(var_ref, out_ref):
    qq = pl.program_id(2)
    out_ref[0, 0, :, :] = var_ref[0, 0, :, pl.ds(128 * (15 - qq), K)]


@jax.jit
def _bias_grid(relative_attention_bias):
    mesh = plsc.VectorSubcoreMesh(
        core_axis_name="c", subcore_axis_name="s", num_cores=2, num_subcores=16
    )
    var_flat = pl.kernel(
        _sc_body,
        out_type=jax.ShapeDtypeStruct((H * NPHASE * NSHIFT * TWP,), jnp.float32),
        mesh=mesh,
        scratch_types=[
            pltpu.VMEM((32, H), jnp.float32),
            pltpu.VMEM((UU * TW,), jnp.float32),
            pltpu.SemaphoreType.DMA,
        ],
        compiler_params=pltpu.CompilerParams(needs_layout_passes=False),
    )(relative_attention_bias)

    var4 = var_flat.reshape(H, NPHASE, NSHIFT, TWP)

    return pl.pallas_call(
        _tc_body,
        grid=(H, NPHASE, GROUPS // NPHASE),
        in_specs=[
            pl.BlockSpec((1, 1, NSHIFT, TWP), lambda hh, mm, qq: (hh, mm, 0, 0)),
        ],
        out_specs=pl.BlockSpec(
            (1, 1, NSHIFT, K), lambda hh, mm, qq: (0, hh, qq * NPHASE + mm, 0)
        ),
        out_shape=jax.ShapeDtypeStruct((1, H, Q, K), jnp.float32),
    )(var4)


def kernel(encoder_hidden, decoder_hidden, relative_attention_bias):
    return _bias_grid(relative_attention_bias)


# speedup vs baseline: 6.5916x; 6.2917x over previous
import jax
import jax.numpy as jnp
from jax import lax
from jax.experimental import pallas as pl
from jax.experimental.pallas import tpu as pltpu
import jax.experimental.pallas.tpu_sc as plsc

Q = 2048
K = 2048
H = 16
NSHIFT = 8
NPHASE = 16
TWC = 256
TW = TWC * 16
TWP = 3968
UU = 4
GROUPS = Q // NSHIFT

_THRESH = (12, 16, 23, 32, 46, 64, 91)


def _sc_body(bias_hbm, var_hbm, bias_v, tbl_v, sem):
    c = lax.axis_index("c")
    s = lax.axis_index("s")
    h = s

    pltpu.sync_copy(bias_hbm, bias_v)

    lane = lax.iota(jnp.int32, 16)
    hvec = jnp.full((16,), h, jnp.int32)

    for uu in range(UU):
        ushift = 7 - uu - (Q - 1)

        def build(t, carry, _ushift=ushift, _uu=uu):
            d = t * 16 + (_ushift - 4 * c) + lane
            pos = d > 0
            a = jnp.abs(d)
            large = jnp.full((16,), 8, jnp.int32)
            for idx, thr in enumerate(_THRESH):
                large = jnp.where(a >= thr, 9 + idx, large)
            sub = jnp.where(a < 8, a, large)
            bkt = jnp.where(pos, sub + 16, sub)
            vals = plsc.load_gather(bias_v, [bkt, hvec])
            tbl_v[pl.ds((_uu * TWC + t) * 16, 16)] = vals
            return carry

        lax.fori_loop(0, TWC, build, 0)

    for uu in range(UU):
        for m in range(NPHASE):
            src = tbl_v.at[pl.ds(uu * TW + (120 - 8 * m), TWP)]
            dst_off = ((h * NPHASE + m) * NSHIFT + 4 * c + uu) * TWP
            pltpu.async_copy(src, var_hbm.at[pl.ds(dst_off, TWP)], sem)
    for _ in range(UU * NPHASE):
        pltpu.make_async_copy(
            var_hbm.at[pl.ds(0, TWP)], tbl_v.at[pl.ds(0, TWP)], sem
        ).wait()


def _tc_body(var_ref, out_ref):
    qq = pl.program_id(1)
    col = 128 * (15 - qq)
    for g in range(NPHASE):
        out_ref[0, 0, 8 * g:8 * g + 8, :] = var_ref[0, g, :, pl.ds(col, K)]


@jax.jit
def _bias_grid(relative_attention_bias):
    mesh = plsc.VectorSubcoreMesh(
        core_axis_name="c", subcore_axis_name="s", num_cores=2, num_subcores=16
    )
    var_flat = pl.kernel(
        _sc_body,
        out_type=jax.ShapeDtypeStruct((H * NPHASE * NSHIFT * TWP,), jnp.float32),
        mesh=mesh,
        scratch_types=[
            pltpu.VMEM((32, H), jnp.float32),
            pltpu.VMEM((UU * TW,), jnp.float32),
            pltpu.SemaphoreType.DMA,
        ],
        compiler_params=pltpu.CompilerParams(needs_layout_passes=False),
    )(relative_attention_bias)

    var4 = var_flat.reshape(H, NPHASE, NSHIFT, TWP)

    return pl.pallas_call(
        _tc_body,
        grid=(H, GROUPS // NPHASE),
        in_specs=[
            pl.BlockSpec((1, NPHASE, NSHIFT, TWP), lambda hh, qq: (hh, 0, 0, 0)),
        ],
        out_specs=pl.BlockSpec(
            (1, 1, NPHASE * NSHIFT, K), lambda hh, qq: (0, hh, qq, 0)
        ),
        out_shape=jax.ShapeDtypeStruct((1, H, Q, K), jnp.float32),
    )(var4)


def kernel(encoder_hidden, decoder_hidden, relative_attention_bias):
    return _bias_grid(relative_attention_bias)


# speedup vs baseline: 8.3912x; 1.2730x over previous
import jax
import jax.numpy as jnp
from jax import lax
from jax.experimental import pallas as pl
from jax.experimental.pallas import tpu as pltpu
import jax.experimental.pallas.tpu_sc as plsc

Q = 2048
K = 2048
H = 16
NSHIFT = 8
NPHASE = 16
TWC = 256
TW = TWC * 16
TWP = 3968
UU = 4
GROUPS = Q // NSHIFT

_THRESH = (12, 16, 23, 32, 46, 64, 91)


def _sc_body(bias_hbm, var_hbm, bias_v, tbl_v, sem):
    c = lax.axis_index("c")
    s = lax.axis_index("s")
    h = s

    pltpu.sync_copy(bias_hbm, bias_v)

    lane = lax.iota(jnp.int32, 16)
    hvec = jnp.full((16,), h, jnp.int32)

    for uu in range(UU):
        ushift = 7 - uu - (Q - 1)

        def build(t, carry, _ushift=ushift, _uu=uu):
            d = t * 16 + (_ushift - 4 * c) + lane
            pos = d > 0
            a = jnp.abs(d)
            large = jnp.full((16,), 8, jnp.int32)
            for idx, thr in enumerate(_THRESH):
                large = jnp.where(a >= thr, 9 + idx, large)
            sub = jnp.where(a < 8, a, large)
            bkt = jnp.where(pos, sub + 16, sub)
            vals = plsc.load_gather(bias_v, [bkt, hvec])
            tbl_v[pl.ds((_uu * TWC + t) * 16, 16)] = vals
            return carry

        lax.fori_loop(0, TWC, build, 0)

    for uu in range(UU):
        for m in range(NPHASE):
            src = tbl_v.at[pl.ds(uu * TW + (120 - 8 * m), TWP)]
            dst_off = ((h * NPHASE + m) * NSHIFT + 4 * c + uu) * TWP
            pltpu.async_copy(src, var_hbm.at[pl.ds(dst_off, TWP)], sem)
    for _ in range(UU * NPHASE):
        pltpu.make_async_copy(
            var_hbm.at[pl.ds(0, TWP)], tbl_v.at[pl.ds(0, TWP)], sem
        ).wait()


_DEPTH = 4
_NSTEP = H * (GROUPS // NPHASE)


def _tc_body(var_ref, out_ref, sem):
    h = pl.program_id(0)
    qq = pl.program_id(1)
    step = h * (GROUPS // NPHASE) + qq
    col = 128 * (15 - qq)

    pltpu.async_copy(
        var_ref.at[0, :, pl.ds(col, K)],
        out_ref.at[0, h, pl.ds(128 * qq, 128), :],
        sem,
    )

    @pl.when(step >= _DEPTH)
    def _retire():
        pltpu.make_async_copy(
            var_ref.at[0, :, pl.ds(0, K)],
            out_ref.at[0, 0, pl.ds(0, 128), :],
            sem,
        ).wait()

    @pl.when(step == _NSTEP - 1)
    def _drain():
        for _ in range(_DEPTH):
            pltpu.make_async_copy(
                var_ref.at[0, :, pl.ds(0, K)],
                out_ref.at[0, 0, pl.ds(0, 128), :],
                sem,
            ).wait()


@jax.jit
def _bias_grid(relative_attention_bias):
    mesh = plsc.VectorSubcoreMesh(
        core_axis_name="c", subcore_axis_name="s", num_cores=2, num_subcores=16
    )
    var_flat = pl.kernel(
        _sc_body,
        out_type=jax.ShapeDtypeStruct((H * NPHASE * NSHIFT * TWP,), jnp.float32),
        mesh=mesh,
        scratch_types=[
            pltpu.VMEM((32, H), jnp.float32),
            pltpu.VMEM((UU * TW,), jnp.float32),
            pltpu.SemaphoreType.DMA,
        ],
        compiler_params=pltpu.CompilerParams(needs_layout_passes=False),
    )(relative_attention_bias)

    var3 = var_flat.reshape(H, NPHASE * NSHIFT, TWP)

    return pl.pallas_call(
        _tc_body,
        grid=(H, GROUPS // NPHASE),
        in_specs=[
            pl.BlockSpec((1, NPHASE * NSHIFT, TWP), lambda hh, qq: (hh, 0, 0)),
        ],
        out_specs=pl.BlockSpec(memory_space=pl.ANY),
        out_shape=jax.ShapeDtypeStruct((1, H, Q, K), jnp.float32),
        scratch_shapes=[pltpu.SemaphoreType.DMA],
    )(var3)


def kernel(encoder_hidden, decoder_hidden, relative_attention_bias):
    return _bias_grid(relative_attention_bias)


# speedup vs baseline: 8.6826x; 1.0347x over previous
import jax
import jax.numpy as jnp
from jax import lax
from jax.experimental import pallas as pl
from jax.experimental.pallas import tpu as pltpu
import jax.experimental.pallas.tpu_sc as plsc

Q = 2048
K = 2048
H = 16
NSHIFT = 8
NPHASE = 16
TWC = 256
TW = TWC * 16
TWP = 3968
UU = 4
GROUPS = Q // NSHIFT

_THRESH = (12, 16, 23, 32, 46, 64, 91)


def _sc_body(bias_hbm, var_hbm, bias_v, tbl_v, sem):
    c = lax.axis_index("c")
    s = lax.axis_index("s")
    h = s

    pltpu.sync_copy(bias_hbm, bias_v)

    lane = lax.iota(jnp.int32, 16)
    hvec = jnp.full((16,), h, jnp.int32)

    for uu in range(UU):
        ushift = 7 - uu - (Q - 1)

        def build(t, carry, _ushift=ushift, _uu=uu):
            d = t * 16 + (_ushift - 4 * c) + lane
            pos = d > 0
            a = jnp.abs(d)
            large = jnp.full((16,), 8, jnp.int32)
            for idx, thr in enumerate(_THRESH):
                large = jnp.where(a >= thr, 9 + idx, large)
            sub = jnp.where(a < 8, a, large)
            bkt = jnp.where(pos, sub + 16, sub)
            vals = plsc.load_gather(bias_v, [bkt, hvec])
            tbl_v[pl.ds((_uu * TWC + t) * 16, 16)] = vals
            return carry

        lax.fori_loop(0, TWC, build, 0)

    for uu in range(UU):
        for m in range(NPHASE):
            src = tbl_v.at[pl.ds(uu * TW + (120 - 8 * m), TWP)]
            dst_off = ((h * NPHASE + m) * NSHIFT + 4 * c + uu) * TWP
            pltpu.async_copy(src, var_hbm.at[pl.ds(dst_off, TWP)], sem)
    for _ in range(UU * NPHASE):
        pltpu.make_async_copy(
            var_hbm.at[pl.ds(0, TWP)], tbl_v.at[pl.ds(0, TWP)], sem
        ).wait()


_DEPTH = 8
_NSTEP = H * (GROUPS // NPHASE)


def _tc_body(var_ref, out_ref, sem):
    h = pl.program_id(0)
    qq = pl.program_id(1)
    step = h * (GROUPS // NPHASE) + qq
    col = 128 * (15 - qq)

    pltpu.async_copy(
        var_ref.at[0, :, pl.ds(col, K)],
        out_ref.at[0, h, pl.ds(128 * qq, 128), :],
        sem,
    )

    @pl.when(step >= _DEPTH)
    def _retire():
        pltpu.make_async_copy(
            var_ref.at[0, :, pl.ds(0, K)],
            out_ref.at[0, 0, pl.ds(0, 128), :],
            sem,
        ).wait()

    @pl.when(step == _NSTEP - 1)
    def _drain():
        for _ in range(_DEPTH):
            pltpu.make_async_copy(
                var_ref.at[0, :, pl.ds(0, K)],
                out_ref.at[0, 0, pl.ds(0, 128), :],
                sem,
            ).wait()


@jax.jit
def _bias_grid(relative_attention_bias):
    mesh = plsc.VectorSubcoreMesh(
        core_axis_name="c", subcore_axis_name="s", num_cores=2, num_subcores=16
    )
    var_flat = pl.kernel(
        _sc_body,
        out_type=jax.ShapeDtypeStruct((H * NPHASE * NSHIFT * TWP,), jnp.float32),
        mesh=mesh,
        scratch_types=[
            pltpu.VMEM((32, H), jnp.float32),
            pltpu.VMEM((UU * TW,), jnp.float32),
            pltpu.SemaphoreType.DMA,
        ],
        compiler_params=pltpu.CompilerParams(needs_layout_passes=False),
    )(relative_attention_bias)

    var3 = var_flat.reshape(H, NPHASE * NSHIFT, TWP)

    return pl.pallas_call(
        _tc_body,
        grid=(H, GROUPS // NPHASE),
        in_specs=[
            pl.BlockSpec((1, NPHASE * NSHIFT, TWP), lambda hh, qq: (hh, 0, 0)),
        ],
        out_specs=pl.BlockSpec(memory_space=pl.ANY),
        out_shape=jax.ShapeDtypeStruct((1, H, Q, K), jnp.float32),
        scratch_shapes=[pltpu.SemaphoreType.DMA],
    )(var3)


def kernel(encoder_hidden, decoder_hidden, relative_attention_bias):
    return _bias_grid(relative_attention_bias)


# speedup vs baseline: 8.6966x; 1.0016x over previous
import jax
import jax.numpy as jnp
from jax import lax
from jax.experimental import pallas as pl
from jax.experimental.pallas import tpu as pltpu
import jax.experimental.pallas.tpu_sc as plsc

Q = 2048
K = 2048
H = 16
NSHIFT = 8
NPHASE = 16
TWC = 256
TW = TWC * 16
TWP = 3968
UU = 4
GROUPS = Q // NSHIFT

_THRESH = (12, 16, 23, 32, 46, 64, 91)


def _sc_body(bias_hbm, var_hbm, bias_v, tbl_v, sem):
    c = lax.axis_index("c")
    s = lax.axis_index("s")
    h = s

    pltpu.sync_copy(bias_hbm, bias_v)

    lane = lax.iota(jnp.int32, 16)
    hvec = jnp.full((16,), h, jnp.int32)

    for uu in range(UU):
        ushift = 7 - uu - (Q - 1)

        def build(t, carry, _ushift=ushift, _uu=uu):
            d = t * 16 + (_ushift - 4 * c) + lane
            pos = d > 0
            a = jnp.abs(d)
            large = jnp.full((16,), 8, jnp.int32)
            for idx, thr in enumerate(_THRESH):
                large = jnp.where(a >= thr, 9 + idx, large)
            sub = jnp.where(a < 8, a, large)
            bkt = jnp.where(pos, sub + 16, sub)
            vals = plsc.load_gather(bias_v, [bkt, hvec])
            tbl_v[pl.ds((_uu * TWC + t) * 16, 16)] = vals
            return carry

        lax.fori_loop(0, TWC, build, 0)

    for uu in range(UU):
        for m in range(NPHASE):
            src = tbl_v.at[pl.ds(uu * TW + (120 - 8 * m), TWP)]
            dst_off = ((h * NPHASE + m) * NSHIFT + 4 * c + uu) * TWP
            pltpu.async_copy(src, var_hbm.at[pl.ds(dst_off, TWP)], sem)
    for _ in range(UU * NPHASE):
        pltpu.make_async_copy(
            var_hbm.at[pl.ds(0, TWP)], tbl_v.at[pl.ds(0, TWP)], sem
        ).wait()


_DEPTH = 8
_NSTEP = H * (GROUPS // NPHASE)


def _tc_body(var_ref, out_ref, sem):
    h = pl.program_id(0)
    qq = pl.program_id(1)
    step = h * (GROUPS // NPHASE) + qq
    col = 128 * (15 - qq)

    for half in range(2):
        pltpu.async_copy(
            var_ref.at[0, pl.ds(64 * half, 64), pl.ds(col, K)],
            out_ref.at[0, h, pl.ds(128 * qq + 64 * half, 64), :],
            sem,
        )

    @pl.when(step >= _DEPTH)
    def _retire():
        for _ in range(2):
            pltpu.make_async_copy(
                var_ref.at[0, pl.ds(0, 64), pl.ds(0, K)],
                out_ref.at[0, 0, pl.ds(0, 64), :],
                sem,
            ).wait()

    @pl.when(step == _NSTEP - 1)
    def _drain():
        for _ in range(2 * _DEPTH):
            pltpu.make_async_copy(
                var_ref.at[0, pl.ds(0, 64), pl.ds(0, K)],
                out_ref.at[0, 0, pl.ds(0, 64), :],
                sem,
            ).wait()


@jax.jit
def _bias_grid(relative_attention_bias):
    mesh = plsc.VectorSubcoreMesh(
        core_axis_name="c", subcore_axis_name="s", num_cores=2, num_subcores=16
    )
    var_flat = pl.kernel(
        _sc_body,
        out_type=jax.ShapeDtypeStruct((H * NPHASE * NSHIFT * TWP,), jnp.float32),
        mesh=mesh,
        scratch_types=[
            pltpu.VMEM((32, H), jnp.float32),
            pltpu.VMEM((UU * TW,), jnp.float32),
            pltpu.SemaphoreType.DMA,
        ],
        compiler_params=pltpu.CompilerParams(needs_layout_passes=False),
    )(relative_attention_bias)

    var3 = var_flat.reshape(H, NPHASE * NSHIFT, TWP)

    return pl.pallas_call(
        _tc_body,
        grid=(H, GROUPS // NPHASE),
        in_specs=[
            pl.BlockSpec((1, NPHASE * NSHIFT, TWP), lambda hh, qq: (hh, 0, 0)),
        ],
        out_specs=pl.BlockSpec(memory_space=pl.ANY),
        out_shape=jax.ShapeDtypeStruct((1, H, Q, K), jnp.float32),
        scratch_shapes=[pltpu.SemaphoreType.DMA],
    )(var3)


def kernel(encoder_hidden, decoder_hidden, relative_attention_bias):
    return _bias_grid(relative_attention_bias)
